# baseline (speedup 1.0000x reference)
import functools

import jax
import jax.numpy as jnp
from jax import lax
from jax.experimental import pallas as pl
from jax.experimental.pallas import tpu as pltpu
from jax.experimental.pallas import tpu_sc as plsc

_D = 128
_FLAT = _D * (_D + 1) // 2
_TB = 128
_NSEM = 4
_CHUNK = 512


def _sc_body(flat_hbm, out_hbm, vbuf, mbuf):
    info = plsc.get_sparse_core_info()
    nc = info.num_cores
    w = lax.axis_index("s") * nc + lax.axis_index("c")
    base = pl.multiple_of(8 * w * w + 2 * w - lax.rem(2 * w, 8), 8)
    pltpu.sync_copy(flat_hbm.at[pl.ds(base, _CHUNK)], vbuf)

    lo = _CHUNK * w
    iota16 = lax.broadcasted_iota(jnp.int32, (16,), 0)
    for j in range(_CHUNK // 16):
        p = lo + j * 16 + iota16
        r = lax.shift_right_logical(p, 7)
        c = lax.bitwise_and(p, 127)
        src = lax.shift_right_logical(r * (r + 1), 1) + c - base
        mask = c <= r
        srcc = jnp.clip(src, 0, _CHUNK - 1)
        v = plsc.load_gather(vbuf, [srcc], mask=mask)
        mbuf[pl.ds(j * 16, 16)] = jnp.where(mask, v, 0.0)

    pltpu.sync_copy(mbuf, out_hbm.at[pl.ds(lo, _CHUNK)])


def _sc_scatter(flat):
    mesh = plsc.VectorSubcoreMesh(core_axis_name="c", subcore_axis_name="s")
    run = functools.partial(
        pl.kernel,
        mesh=mesh,
        out_type=jax.ShapeDtypeStruct((_D * _D,), jnp.float32),
        scratch_types=[
            pltpu.VMEM((_CHUNK,), jnp.float32),
            pltpu.VMEM((_CHUNK,), jnp.float32),
        ],
        compiler_params=pltpu.CompilerParams(needs_layout_passes=False),
    )(_sc_body)
    return run(flat)


def _tc_body(m_ref, out_ref, s_ref, buf_ref, sems):
    m = m_ref[...]
    row = lax.broadcasted_iota(jnp.int32, (_D, _D), 0)
    col = lax.broadcasted_iota(jnp.int32, (_D, _D), 1)
    s_ref[...] = jnp.where(row == col, jax.nn.softplus(m), m)

    buf_ref[...] = jnp.broadcast_to(s_ref[...][None, :, :], (_TB, _D, _D))

    n = out_ref.shape[0] // _TB

    def issue(i, _):
        for k in range(_NSEM):
            pltpu.make_async_copy(
                buf_ref, out_ref.at[pl.ds((i * _NSEM + k) * _TB, _TB)],
                sems.at[k]).start()
        return 0

    lax.fori_loop(0, n // _NSEM, issue, 0)

    def drain(i, _):
        for k in range(_NSEM):
            pltpu.make_async_copy(
                buf_ref, out_ref.at[pl.ds(0, _TB)], sems.at[k]).wait()
        return 0

    lax.fori_loop(0, n // _NSEM, drain, 0)


def kernel(sigma2_schedule, t_schedule, flat_noise_sigma_chol):
    Tn = sigma2_schedule.shape[0]
    m = _sc_scatter(flat_noise_sigma_chol).reshape(_D, _D)

    return pl.pallas_call(
        _tc_body,
        in_specs=[pl.BlockSpec((_D, _D), lambda: (0, 0))],
        out_specs=pl.BlockSpec(memory_space=pl.ANY),
        out_shape=jax.ShapeDtypeStruct((Tn, _D, _D), jnp.float32),
        scratch_shapes=[
            pltpu.VMEM((_D, _D), jnp.float32),
            pltpu.VMEM((_TB, _D, _D), jnp.float32),
            pltpu.SemaphoreType.DMA((_NSEM,)),
        ],
    )(m)


# speedup vs baseline: 1.0011x; 1.0011x over previous
import functools

import jax
import jax.numpy as jnp
from jax import lax
from jax.experimental import pallas as pl
from jax.experimental.pallas import tpu as pltpu
from jax.experimental.pallas import tpu_sc as plsc

_D = 128
_FLAT = _D * (_D + 1) // 2
_TB = 128
_NSEM = 4
_CHUNK = 512
_SPLIT = 1024


def _sc_body(flat_hbm, out_hbm, vbuf, mbuf):
    info = plsc.get_sparse_core_info()
    nc = info.num_cores
    w = lax.axis_index("s") * nc + lax.axis_index("c")
    base = pl.multiple_of(8 * w * w + 2 * w - lax.rem(2 * w, 8), 8)
    pltpu.sync_copy(flat_hbm.at[pl.ds(base, _CHUNK)], vbuf)

    lo = _CHUNK * w
    iota16 = lax.broadcasted_iota(jnp.int32, (16,), 0)
    for j in range(_CHUNK // 16):
        p = lo + j * 16 + iota16
        r = lax.shift_right_logical(p, 7)
        c = lax.bitwise_and(p, 127)
        src = lax.shift_right_logical(r * (r + 1), 1) + c - base
        mask = c <= r
        srcc = jnp.clip(src, 0, _CHUNK - 1)
        v = plsc.load_gather(vbuf, [srcc], mask=mask)
        mbuf[pl.ds(j * 16, 16)] = jnp.where(mask, v, 0.0)

    pltpu.sync_copy(mbuf, out_hbm.at[pl.ds(lo, _CHUNK)])


def _sc_scatter(flat):
    mesh = plsc.VectorSubcoreMesh(core_axis_name="c", subcore_axis_name="s")
    run = functools.partial(
        pl.kernel,
        mesh=mesh,
        out_type=jax.ShapeDtypeStruct((_D * _D,), jnp.float32),
        scratch_types=[
            pltpu.VMEM((_CHUNK,), jnp.float32),
            pltpu.VMEM((_CHUNK,), jnp.float32),
        ],
        compiler_params=pltpu.CompilerParams(needs_layout_passes=False),
    )(_sc_body)
    return run(flat)


def _stream_out(buf_ref, out_ref, sems, t0, t1):
    n = (t1 - t0) // _TB

    def issue(i, _):
        for k in range(_NSEM):
            pltpu.make_async_copy(
                buf_ref,
                out_ref.at[pl.ds(t0 + (i * _NSEM + k) * _TB, _TB)],
                sems.at[k]).start()
        return 0

    lax.fori_loop(0, n // _NSEM, issue, 0)

    def drain(i, _):
        for k in range(_NSEM):
            pltpu.make_async_copy(
                buf_ref, out_ref.at[pl.ds(t0, _TB)], sems.at[k]).wait()
        return 0

    lax.fori_loop(0, n // _NSEM, drain, 0)


def _tc_a_body(flat_ref, out_ref, m_ref, buf_ref, sems):
    col = lax.broadcasted_iota(jnp.int32, (1, _D), 1)
    for r in range(_D):
        s = r * (r + 1) // 2
        vals = flat_ref[0, s:s + _D].reshape(1, _D)
        row = jnp.where(
            col < r, vals,
            jnp.where(col == r, jax.nn.softplus(vals), 0.0))
        m_ref[r:r + 1, :] = row

    buf_ref[...] = jnp.broadcast_to(m_ref[...][None, :, :], (_TB, _D, _D))
    _stream_out(buf_ref, out_ref, sems, 0, _SPLIT)


def _tc_b_body(acc_ref, m_ref, out_ref, s_ref, buf_ref, sems):
    del acc_ref
    m = m_ref[...]
    row = lax.broadcasted_iota(jnp.int32, (_D, _D), 0)
    col = lax.broadcasted_iota(jnp.int32, (_D, _D), 1)
    s_ref[...] = jnp.where(row == col, jax.nn.softplus(m), m)

    buf_ref[...] = jnp.broadcast_to(s_ref[...][None, :, :], (_TB, _D, _D))
    _stream_out(buf_ref, out_ref, sems, _SPLIT, out_ref.shape[0])


def kernel(sigma2_schedule, t_schedule, flat_noise_sigma_chol):
    Tn = sigma2_schedule.shape[0]
    m_sc = _sc_scatter(flat_noise_sigma_chol).reshape(_D, _D)

    flat2 = jnp.concatenate(
        [flat_noise_sigma_chol,
         jnp.zeros((_D,), jnp.float32)]).reshape(1, _FLAT + _D)

    part = pl.pallas_call(
        _tc_a_body,
        in_specs=[pl.BlockSpec((1, _FLAT + _D), lambda: (0, 0))],
        out_specs=pl.BlockSpec(memory_space=pl.ANY),
        out_shape=jax.ShapeDtypeStruct((Tn, _D, _D), jnp.float32),
        scratch_shapes=[
            pltpu.VMEM((_D, _D), jnp.float32),
            pltpu.VMEM((_TB, _D, _D), jnp.float32),
            pltpu.SemaphoreType.DMA((_NSEM,)),
        ],
    )(flat2)

    return pl.pallas_call(
        _tc_b_body,
        in_specs=[
            pl.BlockSpec(memory_space=pl.ANY),
            pl.BlockSpec((_D, _D), lambda: (0, 0)),
        ],
        out_specs=pl.BlockSpec(memory_space=pl.ANY),
        out_shape=jax.ShapeDtypeStruct((Tn, _D, _D), jnp.float32),
        input_output_aliases={0: 0},
        scratch_shapes=[
            pltpu.VMEM((_D, _D), jnp.float32),
            pltpu.VMEM((_TB, _D, _D), jnp.float32),
            pltpu.SemaphoreType.DMA((_NSEM,)),
        ],
    )(part, m_sc)
